# baseline (speedup 1.0000x reference)
import jax
import jax.numpy as jnp
from jax.experimental import pallas as pl
from jax.experimental.pallas import tpu as pltpu

_N, _D, _K = 8192, 64, 1024
_BLK = 256
_NBLK = _N // _BLK
_MAX_ITERS = 40
_W = 4 * _D


def _persistent_body(thr_ref, data_ref, means_ref, out_ref,
                     acc_ref, x2_ref, hml_ref):
    out_ref[...] = means_ref[...]
    data = data_ref[...]
    x2_ref[...] = jnp.sum(data * data, axis=1, keepdims=True)
    hi = data.astype(jnp.bfloat16)
    r1 = data - hi.astype(jnp.float32)
    mid = r1.astype(jnp.bfloat16)
    r2 = r1 - mid.astype(jnp.float32)
    hml_ref[:, 0 * _D:1 * _D] = hi
    hml_ref[:, 1 * _D:2 * _D] = mid
    hml_ref[:, 2 * _D:3 * _D] = r2.astype(jnp.bfloat16)
    hml_ref[:, 3 * _D:4 * _D] = jnp.ones((_N, _D), jnp.bfloat16)

    def one_iter():
        m = out_ref[...]
        m2 = jnp.sum(m * m, axis=1)[None, :]
        acc_ref[...] = jnp.zeros_like(acc_ref)

        def blk(b, carry):
            x = data_ref[pl.ds(b * _BLK, _BLK), :]
            x2 = x2_ref[pl.ds(b * _BLK, _BLK), :]
            xm = jax.lax.dot_general(x, m, (((1,), (1,)), ((), ())),
                                     preferred_element_type=jnp.float32)
            d2 = x2 - 2.0 * xm + m2
            labels = jnp.argmin(d2, axis=1)
            onehot = (labels[:, None] ==
                      jax.lax.broadcasted_iota(jnp.int32, (_BLK, _K), 1)
                      ).astype(jnp.bfloat16)
            hml = hml_ref[pl.ds(b * _BLK, _BLK), :]
            acc_ref[...] += jax.lax.dot_general(
                onehot, hml, (((0,), (0,)), ((), ())),
                preferred_element_type=jnp.float32)
            return carry

        jax.lax.fori_loop(0, _NBLK, blk, 0)
        acc = acc_ref[...]
        sums = (acc[:, 0 * _D:1 * _D] + acc[:, 1 * _D:2 * _D]
                + acc[:, 2 * _D:3 * _D])
        counts = acc[:, 3 * _D:3 * _D + 1]
        m_new = jnp.where(counts > 0.0,
                          sums / jnp.maximum(counts, 1.0),
                          m)
        out_ref[...] = m_new
        diff = m - m_new
        return jnp.max(jnp.sum(diff * diff, axis=1))

    thr = thr_ref[0, 0]

    def cond_fun(carry):
        delta, it = carry
        return (delta > thr) & (it < _MAX_ITERS)

    def body_fun(carry):
        _, it = carry
        delta = one_iter()
        return delta, it + 1

    jax.lax.while_loop(cond_fun, body_fun,
                       (thr + 1.0, jnp.asarray(0, jnp.int32)))


def kernel(Data, means, threshold):
    thr = jnp.asarray(threshold, jnp.float32).reshape(1, 1)
    means_final = pl.pallas_call(
        _persistent_body,
        in_specs=[
            pl.BlockSpec(memory_space=pltpu.SMEM),
            pl.BlockSpec(memory_space=pltpu.VMEM),
            pl.BlockSpec(memory_space=pltpu.VMEM),
        ],
        out_specs=pl.BlockSpec(memory_space=pltpu.VMEM),
        out_shape=jax.ShapeDtypeStruct((_K, _D), jnp.float32),
        scratch_shapes=[
            pltpu.VMEM((_K, _W), jnp.float32),
            pltpu.VMEM((_N, 1), jnp.float32),
            pltpu.VMEM((_N, _W), jnp.bfloat16),
        ],
    )(thr, Data, means)
    return means_final


# speedup vs baseline: 1.2220x; 1.2220x over previous
import jax
import jax.numpy as jnp
from jax.experimental import pallas as pl
from jax.experimental.pallas import tpu as pltpu

_N, _D, _K = 8192, 64, 1024
_BLK = 512
_NBLK = _N // _BLK
_MAX_ITERS = 40
_W = 4 * _D


def _persistent_body(thr_ref, data_ref, means_ref, out_ref,
                     acc_ref, acc2_ref, x2_ref, hml_ref):
    out_ref[...] = means_ref[...]
    data = data_ref[...]
    x2_ref[...] = jnp.sum(data * data, axis=1, keepdims=True)
    hi = data.astype(jnp.bfloat16)
    r1 = data - hi.astype(jnp.float32)
    mid = r1.astype(jnp.bfloat16)
    r2 = r1 - mid.astype(jnp.float32)
    hml_ref[:, 0 * _D:1 * _D] = hi
    hml_ref[:, 1 * _D:2 * _D] = mid
    hml_ref[:, 2 * _D:3 * _D] = r2.astype(jnp.bfloat16)
    hml_ref[:, 3 * _D:4 * _D] = jnp.ones((_N, _D), jnp.bfloat16)

    def one_iter():
        m = out_ref[...]
        m2 = jnp.sum(m * m, axis=1)[None, :]
        acc_ref[...] = jnp.zeros_like(acc_ref)
        acc2_ref[...] = jnp.zeros_like(acc2_ref)

        def half_blk(b, a_ref):
            x = data_ref[pl.ds(b * _BLK, _BLK), :]
            x2 = x2_ref[pl.ds(b * _BLK, _BLK), :]
            xm = jax.lax.dot_general(x, m, (((1,), (1,)), ((), ())),
                                     preferred_element_type=jnp.float32)
            d2 = x2 - 2.0 * xm + m2
            labels = jnp.argmin(d2, axis=1)
            onehot = (labels[:, None] ==
                      jax.lax.broadcasted_iota(jnp.int32, (_BLK, _K), 1)
                      ).astype(jnp.bfloat16)
            hml = hml_ref[pl.ds(b * _BLK, _BLK), :]
            a_ref[...] += jax.lax.dot_general(
                onehot, hml, (((0,), (0,)), ((), ())),
                preferred_element_type=jnp.float32)

        def blk(i, carry):
            half_blk(2 * i, acc_ref)
            half_blk(2 * i + 1, acc2_ref)
            return carry

        jax.lax.fori_loop(0, _NBLK // 2, blk, 0)
        acc = acc_ref[...] + acc2_ref[...]
        sums = (acc[:, 0 * _D:1 * _D] + acc[:, 1 * _D:2 * _D]
                + acc[:, 2 * _D:3 * _D])
        counts = acc[:, 3 * _D:3 * _D + 1]
        m_new = jnp.where(counts > 0.0,
                          sums / jnp.maximum(counts, 1.0),
                          m)
        out_ref[...] = m_new
        diff = m - m_new
        return jnp.max(jnp.sum(diff * diff, axis=1))

    thr = thr_ref[0, 0]

    def cond_fun(carry):
        delta, it = carry
        return (delta > thr) & (it < _MAX_ITERS)

    def body_fun(carry):
        _, it = carry
        delta = one_iter()
        return delta, it + 1

    jax.lax.while_loop(cond_fun, body_fun,
                       (thr + 1.0, jnp.asarray(0, jnp.int32)))


def kernel(Data, means, threshold):
    thr = jnp.asarray(threshold, jnp.float32).reshape(1, 1)
    means_final = pl.pallas_call(
        _persistent_body,
        in_specs=[
            pl.BlockSpec(memory_space=pltpu.SMEM),
            pl.BlockSpec(memory_space=pltpu.VMEM),
            pl.BlockSpec(memory_space=pltpu.VMEM),
        ],
        out_specs=pl.BlockSpec(memory_space=pltpu.VMEM),
        out_shape=jax.ShapeDtypeStruct((_K, _D), jnp.float32),
        scratch_shapes=[
            pltpu.VMEM((_K, _W), jnp.float32),
            pltpu.VMEM((_K, _W), jnp.float32),
            pltpu.VMEM((_N, 1), jnp.float32),
            pltpu.VMEM((_N, _W), jnp.bfloat16),
        ],
    )(thr, Data, means)
    return means_final


# speedup vs baseline: 1.3176x; 1.0782x over previous
import jax
import jax.numpy as jnp
from jax.experimental import pallas as pl
from jax.experimental.pallas import tpu as pltpu

_N, _D, _K = 8192, 64, 1024
_BLK = 512
_NBLK = _N // _BLK
_MAX_ITERS = 40
_W = 4 * _D


def _persistent_body(thr_ref, data_ref, means_ref, out_ref,
                     acc_ref, x2_ref, hml_ref):
    out_ref[...] = means_ref[...]
    data = data_ref[...]
    x2_ref[...] = jnp.sum(data * data, axis=1, keepdims=True)
    hi = data.astype(jnp.bfloat16)
    r1 = data - hi.astype(jnp.float32)
    mid = r1.astype(jnp.bfloat16)
    r2 = r1 - mid.astype(jnp.float32)
    hml_ref[:, 0 * _D:1 * _D] = hi
    hml_ref[:, 1 * _D:2 * _D] = mid
    hml_ref[:, 2 * _D:3 * _D] = r2.astype(jnp.bfloat16)
    hml_ref[:, 3 * _D:4 * _D] = jnp.ones((_N, _D), jnp.bfloat16)

    def one_iter():
        m = out_ref[...]
        m2 = jnp.sum(m * m, axis=1)[None, :]
        acc_ref[...] = jnp.zeros_like(acc_ref)

        def blk(b, carry):
            x = data_ref[pl.ds(b * _BLK, _BLK), :]
            x2 = x2_ref[pl.ds(b * _BLK, _BLK), :]
            xm = jax.lax.dot_general(x, m, (((1,), (1,)), ((), ())),
                                     preferred_element_type=jnp.float32)
            d2 = x2 - 2.0 * xm + m2
            labels = jnp.argmin(d2, axis=1)
            onehot = (labels[:, None] ==
                      jax.lax.broadcasted_iota(jnp.int32, (_BLK, _K), 1)
                      ).astype(jnp.bfloat16)
            hml = hml_ref[pl.ds(b * _BLK, _BLK), :]
            acc_ref[...] += jax.lax.dot_general(
                hml, onehot, (((0,), (0,)), ((), ())),
                preferred_element_type=jnp.float32)
            return carry

        jax.lax.fori_loop(0, _NBLK, blk, 0)
        acc = acc_ref[...]
        sums_t = (acc[0 * _D:1 * _D, :] + acc[1 * _D:2 * _D, :]
                  + acc[2 * _D:3 * _D, :])
        sums = sums_t.T
        counts = acc[3 * _D:3 * _D + 1, :].T
        m_new = jnp.where(counts > 0.0,
                          sums / jnp.maximum(counts, 1.0),
                          m)
        out_ref[...] = m_new
        diff = m - m_new
        return jnp.max(jnp.sum(diff * diff, axis=1))

    thr = thr_ref[0, 0]

    def cond_fun(carry):
        delta, it = carry
        return (delta > thr) & (it < _MAX_ITERS)

    def body_fun(carry):
        _, it = carry
        delta = one_iter()
        return delta, it + 1

    jax.lax.while_loop(cond_fun, body_fun,
                       (thr + 1.0, jnp.asarray(0, jnp.int32)))


def kernel(Data, means, threshold):
    thr = jnp.asarray(threshold, jnp.float32).reshape(1, 1)
    means_final = pl.pallas_call(
        _persistent_body,
        in_specs=[
            pl.BlockSpec(memory_space=pltpu.SMEM),
            pl.BlockSpec(memory_space=pltpu.VMEM),
            pl.BlockSpec(memory_space=pltpu.VMEM),
        ],
        out_specs=pl.BlockSpec(memory_space=pltpu.VMEM),
        out_shape=jax.ShapeDtypeStruct((_K, _D), jnp.float32),
        scratch_shapes=[
            pltpu.VMEM((_W, _K), jnp.float32),
            pltpu.VMEM((_N, 1), jnp.float32),
            pltpu.VMEM((_N, _W), jnp.bfloat16),
        ],
    )(thr, Data, means)
    return means_final


# speedup vs baseline: 1.4898x; 1.1307x over previous
import jax
import jax.numpy as jnp
from jax.experimental import pallas as pl
from jax.experimental.pallas import tpu as pltpu

_N, _D, _K = 8192, 64, 1024
_BLK = 512
_NBLK = _N // _BLK
_MAX_ITERS = 40
_W = 4 * _D


def _persistent_body(thr_ref, data_ref, means_ref, out_ref,
                     acc_ref, x2_ref, hml_ref, oh_ref):
    out_ref[...] = means_ref[...]
    data = data_ref[...]
    x2_ref[...] = jnp.sum(data * data, axis=1, keepdims=True)
    hi = data.astype(jnp.bfloat16)
    r1 = data - hi.astype(jnp.float32)
    mid = r1.astype(jnp.bfloat16)
    r2 = r1 - mid.astype(jnp.float32)
    hml_ref[:, 0 * _D:1 * _D] = hi
    hml_ref[:, 1 * _D:2 * _D] = mid
    hml_ref[:, 2 * _D:3 * _D] = r2.astype(jnp.bfloat16)
    hml_ref[:, 3 * _D:4 * _D] = jnp.ones((_N, _D), jnp.bfloat16)

    def one_iter():
        m = out_ref[...]
        m2 = jnp.sum(m * m, axis=1)[None, :]
        acc_ref[...] = jnp.zeros_like(acc_ref)

        def assign(b):
            x = data_ref[pl.ds(b * _BLK, _BLK), :]
            x2 = x2_ref[pl.ds(b * _BLK, _BLK), :]
            xm = jax.lax.dot_general(x, m, (((1,), (1,)), ((), ())),
                                     preferred_element_type=jnp.float32)
            d2 = x2 - 2.0 * xm + m2
            labels = jnp.argmin(d2, axis=1)
            oh_ref[b % 2] = (labels[:, None] ==
                             jax.lax.broadcasted_iota(
                                 jnp.int32, (_BLK, _K), 1)
                             ).astype(jnp.bfloat16)

        def reduce(b):
            hml = hml_ref[pl.ds(b * _BLK, _BLK), :]
            acc_ref[...] += jax.lax.dot_general(
                hml, oh_ref[b % 2], (((0,), (0,)), ((), ())),
                preferred_element_type=jnp.float32)

        assign(0)

        def blk(b, carry):
            reduce(b - 1)
            assign(b)
            return carry

        jax.lax.fori_loop(1, _NBLK, blk, 0)
        reduce(_NBLK - 1)
        acc = acc_ref[...]
        sums_t = (acc[0 * _D:1 * _D, :] + acc[1 * _D:2 * _D, :]
                  + acc[2 * _D:3 * _D, :])
        sums = sums_t.T
        counts = acc[3 * _D:3 * _D + 1, :].T
        m_new = jnp.where(counts > 0.0,
                          sums / jnp.maximum(counts, 1.0),
                          m)
        out_ref[...] = m_new
        diff = m - m_new
        return jnp.max(jnp.sum(diff * diff, axis=1))

    thr = thr_ref[0, 0]

    def cond_fun(carry):
        delta, it = carry
        return (delta > thr) & (it < _MAX_ITERS)

    def body_fun(carry):
        _, it = carry
        delta = one_iter()
        return delta, it + 1

    jax.lax.while_loop(cond_fun, body_fun,
                       (thr + 1.0, jnp.asarray(0, jnp.int32)))


def kernel(Data, means, threshold):
    thr = jnp.asarray(threshold, jnp.float32).reshape(1, 1)
    means_final = pl.pallas_call(
        _persistent_body,
        in_specs=[
            pl.BlockSpec(memory_space=pltpu.SMEM),
            pl.BlockSpec(memory_space=pltpu.VMEM),
            pl.BlockSpec(memory_space=pltpu.VMEM),
        ],
        out_specs=pl.BlockSpec(memory_space=pltpu.VMEM),
        out_shape=jax.ShapeDtypeStruct((_K, _D), jnp.float32),
        scratch_shapes=[
            pltpu.VMEM((_W, _K), jnp.float32),
            pltpu.VMEM((_N, 1), jnp.float32),
            pltpu.VMEM((_N, _W), jnp.bfloat16),
            pltpu.VMEM((2, _BLK, _K), jnp.bfloat16),
        ],
    )(thr, Data, means)
    return means_final


# speedup vs baseline: 1.5844x; 1.0635x over previous
import jax
import jax.numpy as jnp
from jax.experimental import pallas as pl
from jax.experimental.pallas import tpu as pltpu

_N, _D, _K = 8192, 64, 1024
_BLK = 512
_NBLK = _N // _BLK
_MAX_ITERS = 40
_W = 4 * _D


def _persistent_body(thr_ref, data_ref, means_ref, out_ref,
                     acc_ref, x2_ref, hml_ref, oh_ref):
    out_ref[...] = means_ref[...]
    data = data_ref[...]
    x2_ref[...] = jnp.sum(data * data, axis=1, keepdims=True)
    hi = data.astype(jnp.bfloat16)
    r1 = data - hi.astype(jnp.float32)
    mid = r1.astype(jnp.bfloat16)
    r2 = r1 - mid.astype(jnp.float32)
    hml_ref[:, 0 * _D:1 * _D] = hi
    hml_ref[:, 1 * _D:2 * _D] = mid
    hml_ref[:, 2 * _D:3 * _D] = r2.astype(jnp.bfloat16)
    hml_ref[:, 3 * _D:4 * _D] = jnp.ones((_N, _D), jnp.bfloat16)

    def one_iter():
        m = out_ref[...]
        m2 = jnp.sum(m * m, axis=1)[None, :]

        def assign(b):
            x = data_ref[pl.ds(b * _BLK, _BLK), :]
            x2 = x2_ref[pl.ds(b * _BLK, _BLK), :]
            xm = jax.lax.dot_general(x, m, (((1,), (1,)), ((), ())),
                                     preferred_element_type=jnp.float32)
            d2 = x2 - 2.0 * xm + m2
            labels = jnp.argmin(d2, axis=1)
            oh_ref[b % 4] = (labels[:, None] ==
                             jax.lax.broadcasted_iota(
                                 jnp.int32, (_BLK, _K), 1)
                             ).astype(jnp.bfloat16)

        def reduce(b, init=False):
            hml = hml_ref[pl.ds(b * _BLK, _BLK), :]
            d = jax.lax.dot_general(
                hml, oh_ref[b % 4], (((0,), (0,)), ((), ())),
                preferred_element_type=jnp.float32)
            if init:
                acc_ref[...] = d
            else:
                acc_ref[...] += d

        assign(0)
        assign(1)
        reduce(0, init=True)
        reduce(1)
        assign(2)
        assign(3)

        def blk(s, carry):
            reduce(2 * s - 2)
            reduce(2 * s - 1)
            assign(2 * s)
            assign(2 * s + 1)
            return carry

        jax.lax.fori_loop(2, _NBLK // 2, blk, 0)
        reduce(_NBLK - 2)
        reduce(_NBLK - 1)
        acc = acc_ref[...]
        sums_t = (acc[0 * _D:1 * _D, :] + acc[1 * _D:2 * _D, :]
                  + acc[2 * _D:3 * _D, :])
        sums = sums_t.T
        counts = acc[3 * _D:3 * _D + 1, :].T
        m_new = jnp.where(counts > 0.0,
                          sums / jnp.maximum(counts, 1.0),
                          m)
        out_ref[...] = m_new
        diff = m - m_new
        return jnp.max(jnp.sum(diff * diff, axis=1))

    thr = thr_ref[0, 0]

    def cond_fun(carry):
        delta, it = carry
        return (delta > thr) & (it < _MAX_ITERS)

    def body_fun(carry):
        _, it = carry
        delta = one_iter()
        return delta, it + 1

    jax.lax.while_loop(cond_fun, body_fun,
                       (thr + 1.0, jnp.asarray(0, jnp.int32)))


def kernel(Data, means, threshold):
    thr = jnp.asarray(threshold, jnp.float32).reshape(1, 1)
    means_final = pl.pallas_call(
        _persistent_body,
        in_specs=[
            pl.BlockSpec(memory_space=pltpu.SMEM),
            pl.BlockSpec(memory_space=pltpu.VMEM),
            pl.BlockSpec(memory_space=pltpu.VMEM),
        ],
        out_specs=pl.BlockSpec(memory_space=pltpu.VMEM),
        out_shape=jax.ShapeDtypeStruct((_K, _D), jnp.float32),
        scratch_shapes=[
            pltpu.VMEM((_W, _K), jnp.float32),
            pltpu.VMEM((_N, 1), jnp.float32),
            pltpu.VMEM((_N, _W), jnp.bfloat16),
            pltpu.VMEM((4, _BLK, _K), jnp.bfloat16),
        ],
    )(thr, Data, means)
    return means_final


# speedup vs baseline: 2.0347x; 1.2842x over previous
import jax
import jax.numpy as jnp
from jax.experimental import pallas as pl
from jax.experimental.pallas import tpu as pltpu

_N, _D, _K = 8192, 64, 1024
_BLK = 512
_NBLK = _N // _BLK
_MAX_ITERS = 40
_W = 4 * _D


def _persistent_body(thr_ref, data_ref, means_ref, out_ref,
                     acc_ref, x2_ref, hml_ref, oh_ref):
    out_ref[...] = means_ref[...]
    data = data_ref[...]
    x2_ref[...] = jnp.sum(data * data, axis=1, keepdims=True)
    hi = data.astype(jnp.bfloat16)
    r1 = data - hi.astype(jnp.float32)
    mid = r1.astype(jnp.bfloat16)
    r2 = r1 - mid.astype(jnp.float32)
    hml_ref[:, 0 * _D:1 * _D] = hi
    hml_ref[:, 1 * _D:2 * _D] = mid
    hml_ref[:, 2 * _D:3 * _D] = r2.astype(jnp.bfloat16)
    hml_ref[:, 3 * _D:4 * _D] = jnp.ones((_N, _D), jnp.bfloat16)

    def one_iter():
        m = out_ref[...]
        m2 = jnp.sum(m * m, axis=1)[None, :]

        def assign(b):
            x = data_ref[pl.ds(b * _BLK, _BLK), :]
            x2 = x2_ref[pl.ds(b * _BLK, _BLK), :]
            xm = jax.lax.dot_general(x, m, (((1,), (1,)), ((), ())),
                                     preferred_element_type=jnp.float32)
            d2 = x2 - 2.0 * xm + m2
            labels = jnp.argmin(d2, axis=1)
            oh_ref[b % 4] = (labels[:, None] ==
                             jax.lax.broadcasted_iota(
                                 jnp.int32, (_BLK, _K), 1)
                             ).astype(jnp.bfloat16)

        def reduce(b, init=False):
            hml = hml_ref[pl.ds(b * _BLK, _BLK), :]
            d = jax.lax.dot_general(
                hml, oh_ref[b % 4], (((0,), (0,)), ((), ())),
                preferred_element_type=jnp.float32)
            if init:
                acc_ref[...] = d
            else:
                acc_ref[...] += d

        for b in range(4):
            assign(b)
        for b in range(_NBLK):
            reduce(b, init=(b == 0))
            if b + 4 < _NBLK:
                assign(b + 4)
        acc = acc_ref[...]
        sums_t = (acc[0 * _D:1 * _D, :] + acc[1 * _D:2 * _D, :]
                  + acc[2 * _D:3 * _D, :])
        sums = sums_t.T
        counts = acc[3 * _D:3 * _D + 1, :].T
        m_new = jnp.where(counts > 0.0,
                          sums / jnp.maximum(counts, 1.0),
                          m)
        out_ref[...] = m_new
        diff = m - m_new
        return jnp.max(jnp.sum(diff * diff, axis=1))

    thr = thr_ref[0, 0]

    def cond_fun(carry):
        delta, it = carry
        return (delta > thr) & (it < _MAX_ITERS)

    def body_fun(carry):
        _, it = carry
        delta = one_iter()
        return delta, it + 1

    jax.lax.while_loop(cond_fun, body_fun,
                       (thr + 1.0, jnp.asarray(0, jnp.int32)))


def kernel(Data, means, threshold):
    thr = jnp.asarray(threshold, jnp.float32).reshape(1, 1)
    means_final = pl.pallas_call(
        _persistent_body,
        in_specs=[
            pl.BlockSpec(memory_space=pltpu.SMEM),
            pl.BlockSpec(memory_space=pltpu.VMEM),
            pl.BlockSpec(memory_space=pltpu.VMEM),
        ],
        out_specs=pl.BlockSpec(memory_space=pltpu.VMEM),
        out_shape=jax.ShapeDtypeStruct((_K, _D), jnp.float32),
        scratch_shapes=[
            pltpu.VMEM((_W, _K), jnp.float32),
            pltpu.VMEM((_N, 1), jnp.float32),
            pltpu.VMEM((_N, _W), jnp.bfloat16),
            pltpu.VMEM((4, _BLK, _K), jnp.bfloat16),
        ],
    )(thr, Data, means)
    return means_final
